# baseline (speedup 1.0000x reference)
import functools

import jax
import jax.numpy as jnp
from jax.experimental import pallas as pl
from jax.experimental.pallas import tpu as pltpu

D = 128
VB = 1000
B = 2000
U = 16


def _transform_kernel(e_ref, w_ref, b_ref, o_ref):
    x = jax.lax.dot_general(
        e_ref[...], w_ref[...],
        dimension_numbers=(((1,), (1,)), ((), ())),
        preferred_element_type=jnp.float32,
    )
    o_ref[...] = (x + b_ref[...]).reshape(e_ref.shape[0], 1, D)


def _scan_kernel(tok_hbm, e2_ref, s_out, tok_smem, x_tile, carry, sem):
    k = pl.program_id(0)
    cp = pltpu.make_async_copy(tok_hbm.at[pl.ds(k, 1)], tok_smem, sem)
    cp.start()
    cp.wait()

    @pl.when(k == 0)
    def _():
        carry[...] = jnp.zeros_like(carry)

    def gather_chunk(c, _):
        base = c * U
        for u in range(U):
            t = tok_smem[0, base + u]
            x_tile[base + u, 0] = e2_ref[t, 0]
        return 0

    jax.lax.fori_loop(0, B // U, gather_chunk, 0)

    x = x_tile[...].reshape(B, D)
    s = x
    sh = 1
    while sh < B:
        s = s + jnp.concatenate(
            [jnp.zeros((sh, D), jnp.float32), s[: B - sh]], axis=0)
        sh *= 2
    s = s + carry[...]
    s_out[...] = s.reshape(B, 1, D)
    carry[...] = s[B - 1 :, :]


def _resolve_kernel(end_hbm, s_blk, s_prev, s_hbm, o_ref,
                    end_smem, gat, acc, sem_i, sem_g):
    k = pl.program_id(0)
    nk = pl.num_programs(0)
    cp = pltpu.make_async_copy(end_hbm.at[pl.ds(k, 1)], end_smem, sem_i)
    cp.start()
    cp.wait()

    @pl.when(k == 0)
    def _():
        acc[...] = jnp.full_like(acc, -jnp.inf)

    def issue_chunk(c, _):
        base = c * U
        for u in range(U):
            e = end_smem[0, base + u]
            pltpu.make_async_copy(s_hbm.at[e], gat.at[base + u], sem_g).start()
        return 0

    jax.lax.fori_loop(0, B // U, issue_chunk, 0)
    pltpu.make_async_copy(s_hbm.at[pl.ds(0, B)], gat, sem_g).wait()

    sb = s_blk[...].reshape(B, D)
    prev_last = s_prev[...].reshape(B, D)[B - 1 :, :]
    prev_last = jnp.where(k == 0, jnp.zeros((1, D), jnp.float32), prev_last)
    sx = jnp.concatenate([prev_last, sb[: B - 1]], axis=0)
    cand = gat[...].reshape(B, D) - sx
    acc[...] = jnp.maximum(acc[...], jnp.max(cand, axis=0, keepdims=True))

    @pl.when(k == nk - 1)
    def _():
        o_ref[...] = acc[...]


def kernel(tokens, subtree_end, embedding_weight, W, b):
    n = tokens.shape[0]
    v = embedding_weight.shape[0]
    vb = VB if v % VB == 0 else v
    assert n % B == 0
    K = n // B

    e2 = pl.pallas_call(
        _transform_kernel,
        grid=(v // vb,),
        in_specs=[
            pl.BlockSpec((vb, D), lambda i: (i, 0)),
            pl.BlockSpec((D, D), lambda i: (0, 0)),
            pl.BlockSpec((1, D), lambda i: (0, 0)),
        ],
        out_specs=pl.BlockSpec((vb, 1, D), lambda i: (i, 0, 0)),
        out_shape=jax.ShapeDtypeStruct((v, 1, D), jnp.float32),
        compiler_params=pltpu.CompilerParams(
            dimension_semantics=("arbitrary",)),
        name="table_transform",
    )(embedding_weight, W, b.reshape(1, D))

    tok2d = tokens.reshape(K, B)
    s = pl.pallas_call(
        _scan_kernel,
        grid=(K,),
        in_specs=[
            pl.BlockSpec(memory_space=pl.ANY),
            pl.BlockSpec(memory_space=pltpu.VMEM),
        ],
        out_specs=pl.BlockSpec((B, 1, D), lambda k: (k, 0, 0)),
        out_shape=jax.ShapeDtypeStruct((n, 1, D), jnp.float32),
        scratch_shapes=[
            pltpu.SMEM((1, B), jnp.int32),
            pltpu.VMEM((B, 1, D), jnp.float32),
            pltpu.VMEM((1, D), jnp.float32),
            pltpu.SemaphoreType.DMA,
        ],
        compiler_params=pltpu.CompilerParams(
            dimension_semantics=("arbitrary",),
            vmem_limit_bytes=48 * 1024 * 1024,
        ),
        name="gather_scan",
    )(tok2d, e2)

    end2d = subtree_end.reshape(K, B)
    out = pl.pallas_call(
        _resolve_kernel,
        grid=(K,),
        in_specs=[
            pl.BlockSpec(memory_space=pl.ANY),
            pl.BlockSpec((B, 1, D), lambda k: (k, 0, 0)),
            pl.BlockSpec((B, 1, D), lambda k: (jnp.maximum(k - 1, 0), 0, 0)),
            pl.BlockSpec(memory_space=pl.ANY),
        ],
        out_specs=pl.BlockSpec((1, D), lambda k: (0, 0)),
        out_shape=jax.ShapeDtypeStruct((1, D), jnp.float32),
        scratch_shapes=[
            pltpu.SMEM((1, B), jnp.int32),
            pltpu.VMEM((B, 1, D), jnp.float32),
            pltpu.VMEM((1, D), jnp.float32),
            pltpu.SemaphoreType.DMA,
            pltpu.SemaphoreType.DMA,
        ],
        compiler_params=pltpu.CompilerParams(
            dimension_semantics=("arbitrary",),
            vmem_limit_bytes=48 * 1024 * 1024,
        ),
        name="resolve_max",
    )(end2d, s, s, s)

    return out[0]


# speedup vs baseline: 1.7784x; 1.7784x over previous
import functools

import jax
import jax.numpy as jnp
from jax.experimental import pallas as pl
from jax.experimental.pallas import tpu as pltpu

D = 128
VB = 1000
B = 2048
G = 256
U = 16
_W_MAX = 16
_BIG = 2**30
_NEG = -3.0e38


def _transform_kernel(e_ref, w_ref, b_ref, o_ref):
    x = jax.lax.dot_general(
        e_ref[...], w_ref[...],
        dimension_numbers=(((1,), (1,)), ((), ())),
        preferred_element_type=jnp.float32,
    )
    o_ref[...] = (x + b_ref[...]).reshape(e_ref.shape[0], 1, D)


def _scan_kernel(tok_hbm, e2_any, l_ref, s_out, tok_smem, x_tile, carry,
                 e2_ref, sem, sem_e):
    k = pl.program_id(0)

    @pl.when(k == 0)
    def _():
        cpe = pltpu.make_async_copy(e2_any, e2_ref, sem_e)
        cpe.start()
        cpe.wait()
        carry[...] = jnp.zeros_like(carry)

    cp = pltpu.make_async_copy(tok_hbm.at[pl.ds(k, 1)], tok_smem, sem)
    cp.start()
    cp.wait()

    def gather_chunk(c, _):
        base = c * U
        for u in range(U):
            t = tok_smem[0, base + u]
            x_tile[base + u, 0] = e2_ref[t, 0]
        return 0

    jax.lax.fori_loop(0, B // U, gather_chunk, 0)

    x = x_tile[...].reshape(B, D)
    xh = x.astype(jnp.bfloat16)
    xl = (x - xh.astype(jnp.float32)).astype(jnp.bfloat16)
    lmat = l_ref[...]
    run = carry[...]
    for g in range(B // G):
        xg_h = xh[g * G:(g + 1) * G]
        xg_l = xl[g * G:(g + 1) * G]
        sg = jax.lax.dot_general(
            lmat, xg_h, dimension_numbers=(((1,), (0,)), ((), ())),
            preferred_element_type=jnp.float32)
        sg = sg + jax.lax.dot_general(
            lmat, xg_l, dimension_numbers=(((1,), (0,)), ((), ())),
            preferred_element_type=jnp.float32)
        sg = sg + run
        s_out[g * G:(g + 1) * G] = sg.reshape(G, 1, D)
        run = sg[G - 1:G] + x[g * G + G - 1:g * G + G]
    carry[...] = run


def _resolve_kernel(end_hbm, end_vec_ref, s_any, o_ref,
                    end_smem, nfar_smem, far_list, win, cand_tile,
                    far_s, sx_far, acc, sem_i, sem_w, sem_f, W, NREAL):
    WB = W * B
    k = pl.program_id(0)
    nk = pl.num_programs(0)

    @pl.when(k == 0)
    def _():
        for j in range(W):
            pltpu.make_async_copy(
                s_any.at[pl.ds(j * B, B)], win.at[pl.ds(j * B, B)],
                sem_w).start()
        acc[...] = jnp.full_like(acc, _NEG)

    @pl.when(k > 0)
    def _():
        jin = jnp.minimum(k + W - 1, nk - 1)
        slot = (k + W - 1) & (W - 1)
        pltpu.make_async_copy(
            s_any.at[pl.ds(jin * B, B)], win.at[pl.ds(slot * B, B)],
            sem_w).start()

    cpi = pltpu.make_async_copy(end_hbm.at[pl.ds(k, 1)], end_smem, sem_i)
    cpi.start()
    cpi.wait()

    ev = end_vec_ref[...].reshape(1, B)
    thresh = k * B + WB - 1
    lane = jax.lax.broadcasted_iota(jnp.int32, (1, B), 1)
    mval0 = jnp.where(ev >= thresh, lane, _BIG)
    nfar_smem[0] = 0

    @pl.when(jnp.min(mval0) < _BIG)
    def _():
        def cond(c):
            return jnp.min(c[0]) < _BIG

        def body(c):
            mval, t = c
            idx = jnp.min(mval)
            e1 = end_smem[0, idx] + 1
            pltpu.make_async_copy(s_any.at[pl.ds(e1, 1)],
                                  far_s.at[pl.ds(t, 1)], sem_f).start()
            sx_far[t, 0] = win[(k * B + idx) & (WB - 1), 0]
            far_list[0, t] = idx
            return jnp.where(lane == idx, _BIG, mval), t + 1

        _, nf = jax.lax.while_loop(cond, body, (mval0, jnp.int32(0)))
        nfar_smem[0] = nf

    @pl.when(k == 0)
    def _():
        pltpu.make_async_copy(s_any.at[pl.ds(0, W * B)], win, sem_w).wait()

    @pl.when(k > 0)
    def _():
        pltpu.make_async_copy(s_any.at[pl.ds(0, B)],
                              win.at[pl.ds(0, B)], sem_w).wait()

    def near_chunk(c, _):
        base = c * U
        for u in range(U):
            e1 = end_smem[0, base + u] + 1
            cand_tile[base + u, 0] = win[e1 & (WB - 1), 0]
        return 0

    jax.lax.fori_loop(0, B // U, near_chunk, 0)

    n_far = nfar_smem[0]

    def poke(t, _):
        cand_tile[far_list[0, t], 0] = jnp.full((D,), _NEG, jnp.float32)
        return 0

    jax.lax.fori_loop(0, n_far, poke, 0)

    slot_k = k & (W - 1)
    base_k = pl.multiple_of(slot_k * B, B)
    sxb = win[pl.ds(base_k, B)].reshape(B, D)
    cand = cand_tile[...].reshape(B, D) - sxb
    rowid = jax.lax.broadcasted_iota(jnp.int32, (B, D), 0) + k * B
    cand = jnp.where(rowid < NREAL, cand, _NEG)
    acc[...] = jnp.maximum(acc[...], jnp.max(cand, axis=0, keepdims=True))

    @pl.when(n_far > 0)
    def _():
        def wait_one(t, _):
            pltpu.make_async_copy(s_any.at[pl.ds(0, 1)],
                                  far_s.at[pl.ds(0, 1)], sem_f).wait()
            return 0

        jax.lax.fori_loop(0, n_far, wait_one, 0)
        fc = far_s[...].reshape(B, D) - sx_far[...].reshape(B, D)
        fid = jax.lax.broadcasted_iota(jnp.int32, (B, D), 0)
        fc = jnp.where(fid < n_far, fc, _NEG)
        acc[...] = jnp.maximum(acc[...], jnp.max(fc, axis=0, keepdims=True))

    @pl.when(k == nk - 1)
    def _():
        o_ref[...] = acc[...]


def kernel(tokens, subtree_end, embedding_weight, W, b):
    n = tokens.shape[0]
    v = embedding_weight.shape[0]
    vb = VB if v % VB == 0 else v
    K = -(-(n + 1) // B)
    npad = K * B
    Wwin = _W_MAX if K >= _W_MAX else 1 << (K.bit_length() - 1)

    e2 = pl.pallas_call(
        _transform_kernel,
        grid=(v // vb,),
        in_specs=[
            pl.BlockSpec((vb, D), lambda i: (i, 0)),
            pl.BlockSpec((D, D), lambda i: (0, 0)),
            pl.BlockSpec((1, D), lambda i: (0, 0)),
        ],
        out_specs=pl.BlockSpec((vb, 1, D), lambda i: (i, 0, 0)),
        out_shape=jax.ShapeDtypeStruct((v, 1, D), jnp.float32),
        compiler_params=pltpu.CompilerParams(
            dimension_semantics=("arbitrary",)),
        name="table_transform",
    )(embedding_weight, W, b.reshape(1, D))

    tok2d = jnp.pad(tokens, (0, npad - n)).reshape(K, B)
    lmat = jnp.tril(jnp.ones((G, G), jnp.bfloat16), -1)
    s = pl.pallas_call(
        _scan_kernel,
        grid=(K,),
        in_specs=[
            pl.BlockSpec(memory_space=pl.ANY),
            pl.BlockSpec(memory_space=pl.ANY),
            pl.BlockSpec(memory_space=pltpu.VMEM),
        ],
        out_specs=pl.BlockSpec((B, 1, D), lambda k: (k, 0, 0)),
        out_shape=jax.ShapeDtypeStruct((npad, 1, D), jnp.float32),
        scratch_shapes=[
            pltpu.SMEM((1, B), jnp.int32),
            pltpu.VMEM((B, 1, D), jnp.float32),
            pltpu.VMEM((1, D), jnp.float32),
            pltpu.VMEM((v, 1, D), jnp.float32),
            pltpu.SemaphoreType.DMA,
            pltpu.SemaphoreType.DMA,
        ],
        compiler_params=pltpu.CompilerParams(
            dimension_semantics=("arbitrary",),
            vmem_limit_bytes=48 * 1024 * 1024,
        ),
        name="gather_scan",
    )(tok2d, e2, lmat)

    end2d = jnp.pad(subtree_end, (0, npad - n)).reshape(K, B)
    end3d = end2d.reshape(K, 1, B)
    out = pl.pallas_call(
        functools.partial(_resolve_kernel, W=Wwin, NREAL=n),
        grid=(K,),
        in_specs=[
            pl.BlockSpec(memory_space=pl.ANY),
            pl.BlockSpec((1, 1, B), lambda k: (k, 0, 0)),
            pl.BlockSpec(memory_space=pl.ANY),
        ],
        out_specs=pl.BlockSpec((1, D), lambda k: (0, 0)),
        out_shape=jax.ShapeDtypeStruct((1, D), jnp.float32),
        scratch_shapes=[
            pltpu.SMEM((1, B), jnp.int32),
            pltpu.SMEM((1,), jnp.int32),
            pltpu.SMEM((1, B), jnp.int32),
            pltpu.VMEM((Wwin * B, 1, D), jnp.float32),
            pltpu.VMEM((B, 1, D), jnp.float32),
            pltpu.VMEM((B, 1, D), jnp.float32),
            pltpu.VMEM((B, 1, D), jnp.float32),
            pltpu.VMEM((1, D), jnp.float32),
            pltpu.SemaphoreType.DMA,
            pltpu.SemaphoreType.DMA,
            pltpu.SemaphoreType.DMA,
        ],
        compiler_params=pltpu.CompilerParams(
            dimension_semantics=("arbitrary",),
            vmem_limit_bytes=56 * 1024 * 1024,
        ),
        name="resolve_max",
    )(end2d, end3d, s)

    return out[0]
